# baseline (speedup 1.0000x reference)
import functools

import jax
import jax.numpy as jnp
from jax import lax
from jax.experimental import pallas as pl
from jax.experimental.pallas import tpu as pltpu, tpu_sc as plsc

B = 16384
F = 26
V = 100000

_INFO = plsc.get_sparse_core_info()
NC = _INFO.num_cores
NS = _INFO.num_subcores
NW = NC * NS
RPW = B // NW
CH = 128
CPF = RPW // CH


def _sc_gather_sum(table_flat, idx_t, bias16):
    mesh = plsc.VectorSubcoreMesh(core_axis_name="c", subcore_axis_name="s")

    @functools.partial(
        pl.kernel,
        out_type=jax.ShapeDtypeStruct((B,), jnp.float32),
        mesh=mesh,
        compiler_params=pltpu.CompilerParams(use_tc_tiling_on_sc=False),
        scratch_types=[
            pltpu.VMEM((F, RPW), jnp.int32),
            pltpu.VMEM((F * RPW,), jnp.float32),
            pltpu.VMEM((16,), jnp.float32),
            pltpu.VMEM((RPW,), jnp.float32),
            pltpu.SemaphoreType.DMA,
            pltpu.SemaphoreType.DMA,
        ],
    )
    def body(table_hbm, idx_hbm, bias_hbm, out_hbm, idx_v, buf, bias_v, out_v,
             sem_idx, sem):
        wid = lax.axis_index("s") * NC + lax.axis_index("c")
        base = wid * RPW
        pltpu.sync_copy(bias_hbm, bias_v)
        for f in range(F):
            pltpu.async_copy(
                idx_hbm.at[f, pl.ds(base, RPW)], idx_v.at[f], sem_idx
            )
        for f in range(F):
            pltpu.make_async_copy(
                idx_hbm.at[f, pl.ds(base, RPW)], idx_v.at[f], sem_idx
            ).wait()

        def chunk(f, c):
            return (
                table_hbm.at[idx_v.at[f, pl.ds(c * CH, CH)]],
                buf.at[pl.ds(f * RPW + c * CH, CH)],
            )

        pending = []
        for f in range(F):
            for c in range(CPF):
                src, dst = chunk(f, c)
                pltpu.async_copy(src, dst, sem)
                pending.append((f, c))
                if len(pending) > 8:
                    so, do = chunk(*pending.pop(0))
                    pltpu.make_async_copy(so, do, sem).wait()
        for fo, co in pending:
            so, do = chunk(fo, co)
            pltpu.make_async_copy(so, do, sem).wait()

        bvec = bias_v[...]
        for g in range(RPW // 16):
            acc = bvec
            for f in range(F):
                acc = acc + buf[pl.ds(f * RPW + g * 16, 16)]
            out_v[pl.ds(g * 16, 16)] = acc
        pltpu.sync_copy(out_v, out_hbm.at[pl.ds(base, RPW)])

    return body(table_flat, idx_t, bias16)


def kernel(x, tables, bias):
    table_flat = jnp.concatenate([tables[f, :, 0] for f in range(F)])
    idx_t = x.T.astype(jnp.int32) + (jnp.arange(F, dtype=jnp.int32) * V)[:, None]
    bias16 = jnp.broadcast_to(bias.astype(jnp.float32), (16,))
    out = _sc_gather_sum(table_flat, idx_t, bias16)
    return out.reshape(B, 1)


# speedup vs baseline: 2.1202x; 2.1202x over previous
import functools

import jax
import jax.numpy as jnp
from jax import lax
from jax.experimental import pallas as pl
from jax.experimental.pallas import tpu as pltpu, tpu_sc as plsc

B = 16384
F = 26
V = 100000

_INFO = plsc.get_sparse_core_info()
NC = _INFO.num_cores
NS = _INFO.num_subcores
NW = NC * NS
RPW = B // NW
CH = 128
CPF = RPW // CH


def _sc_gather_sum(table_rows, idx_t, bias16):
    mesh = plsc.VectorSubcoreMesh(core_axis_name="c", subcore_axis_name="s")

    @functools.partial(
        pl.kernel,
        out_type=jax.ShapeDtypeStruct((B,), jnp.float32),
        mesh=mesh,
        compiler_params=pltpu.CompilerParams(use_tc_tiling_on_sc=False),
        scratch_types=[
            pltpu.VMEM((F, RPW), jnp.int32),
            pltpu.VMEM((F * RPW,), jnp.float32),
            pltpu.VMEM((16,), jnp.float32),
            pltpu.VMEM((RPW,), jnp.float32),
            pltpu.SemaphoreType.DMA,
            pltpu.SemaphoreType.DMA,
        ],
    )
    def body(*refs):
        tabs = refs[:F]
        idx_hbm, bias_hbm, out_hbm, idx_v, buf, bias_v, out_v, sem_idx, sem = refs[F:]
        wid = lax.axis_index("s") * NC + lax.axis_index("c")
        base = wid * RPW
        pltpu.sync_copy(bias_hbm, bias_v)
        for f in range(F):
            pltpu.async_copy(
                idx_hbm.at[f, pl.ds(base, RPW)], idx_v.at[f], sem_idx
            )
        for f in range(F):
            pltpu.make_async_copy(
                idx_hbm.at[f, pl.ds(base, RPW)], idx_v.at[f], sem_idx
            ).wait()

        def chunk(f, c):
            return (
                tabs[f].at[idx_v.at[f, pl.ds(c * CH, CH)]],
                buf.at[pl.ds(f * RPW + c * CH, CH)],
            )

        pending = []
        for f in range(F):
            for c in range(CPF):
                src, dst = chunk(f, c)
                pltpu.async_copy(src, dst, sem)
                pending.append((f, c))
                if len(pending) > 8:
                    so, do = chunk(*pending.pop(0))
                    pltpu.make_async_copy(so, do, sem).wait()
        for fo, co in pending:
            so, do = chunk(fo, co)
            pltpu.make_async_copy(so, do, sem).wait()

        bvec = bias_v[...]
        for g in range(RPW // 16):
            acc = bvec
            for f in range(F):
                acc = acc + buf[pl.ds(f * RPW + g * 16, 16)]
            out_v[pl.ds(g * 16, 16)] = acc
        pltpu.sync_copy(out_v, out_hbm.at[pl.ds(base, RPW)])

    return body(*table_rows, idx_t, bias16)


def kernel(x, tables, bias):
    table_rows = [tables[f, :, 0] for f in range(F)]
    idx_t = x.T.astype(jnp.int32)
    bias16 = jnp.broadcast_to(bias.astype(jnp.float32), (16,))
    out = _sc_gather_sum(table_rows, idx_t, bias16)
    return out.reshape(B, 1)


# speedup vs baseline: 2.3414x; 1.1043x over previous
import functools

import jax
import jax.numpy as jnp
from jax import lax
from jax.experimental import pallas as pl
from jax.experimental.pallas import tpu as pltpu, tpu_sc as plsc

B = 16384
F = 26
V = 100000

_INFO = plsc.get_sparse_core_info()
NC = _INFO.num_cores
NS = _INFO.num_subcores
NW = NC * NS
RPW = B // NW
CH = 128
CPF = RPW // CH


def _sc_gather_sum(table_rows, idx_t, bias16):
    mesh = plsc.VectorSubcoreMesh(core_axis_name="c", subcore_axis_name="s")

    @functools.partial(
        pl.kernel,
        out_type=jax.ShapeDtypeStruct((B,), jnp.float32),
        mesh=mesh,
        compiler_params=pltpu.CompilerParams(use_tc_tiling_on_sc=False),
        scratch_types=[
            pltpu.VMEM((F, RPW), jnp.int32),
            pltpu.VMEM((F * RPW,), jnp.float32),
            pltpu.VMEM((16,), jnp.float32),
            pltpu.VMEM((RPW,), jnp.float32),
            pltpu.SemaphoreType.DMA,
            pltpu.SemaphoreType.DMA,
            pltpu.SemaphoreType.DMA,
            pltpu.SemaphoreType.DMA,
            pltpu.SemaphoreType.DMA,
        ],
    )
    def body(*refs):
        tabs = refs[:F]
        (idx_hbm, bias_hbm, out_hbm, idx_v, buf, bias_v, out_v, sem_idx,
         s0, s1, s2, s3) = refs[F:]
        sems = (s0, s1, s2, s3)
        wid = lax.axis_index("s") * NC + lax.axis_index("c")
        base = wid * RPW
        pltpu.sync_copy(bias_hbm, bias_v)
        for f in range(F):
            pltpu.async_copy(
                idx_hbm.at[f, pl.ds(base, RPW)], idx_v.at[f], sem_idx
            )
        for f in range(F):
            pltpu.make_async_copy(
                idx_hbm.at[f, pl.ds(base, RPW)], idx_v.at[f], sem_idx
            ).wait()

        bvec = bias_v[...]

        def chunk(f, c):
            return (
                tabs[f].at[idx_v.at[f, pl.ds(c * CH, CH)]],
                buf.at[pl.ds(f * RPW + c * CH, CH)],
            )

        def fire(c):
            for f in range(F):
                src, dst = chunk(f, c)
                pltpu.async_copy(src, dst, sems[c])

        def drain(c):
            for f in range(F):
                src, dst = chunk(f, c)
                pltpu.make_async_copy(src, dst, sems[c]).wait()

        def reduce_block(c):
            for v in range(CH // 16):
                acc = bvec
                for f in range(F):
                    acc = acc + buf[pl.ds(f * RPW + c * CH + v * 16, 16)]
                out_v[pl.ds(c * CH + v * 16, 16)] = acc

        fire(0)
        fire(1)
        for c in range(CPF):
            drain(c)
            if c + 2 < CPF:
                fire(c + 2)
            reduce_block(c)
        pltpu.sync_copy(out_v, out_hbm.at[pl.ds(base, RPW)])

    return body(*table_rows, idx_t, bias16)


def kernel(x, tables, bias):
    table_rows = [tables[f, :, 0] for f in range(F)]
    idx_t = x.T.astype(jnp.int32)
    bias16 = jnp.broadcast_to(bias.astype(jnp.float32), (16,))
    out = _sc_gather_sum(table_rows, idx_t, bias16)
    return out.reshape(B, 1)


# speedup vs baseline: 2.5218x; 1.0770x over previous
import functools

import jax
import jax.numpy as jnp
from jax import lax
from jax.experimental import pallas as pl
from jax.experimental.pallas import tpu as pltpu, tpu_sc as plsc

B = 16384
F = 26
V = 100000

_INFO = plsc.get_sparse_core_info()
NC = _INFO.num_cores
NS = _INFO.num_subcores
NW = NC * NS
RPW = B // NW
CH = 128
CPF = RPW // CH
G = 13


def _sc_gather_sum(table_rows, idx_t, bias16):
    mesh = plsc.VectorSubcoreMesh(core_axis_name="c", subcore_axis_name="s")

    @functools.partial(
        pl.kernel,
        out_type=jax.ShapeDtypeStruct((B,), jnp.float32),
        mesh=mesh,
        compiler_params=pltpu.CompilerParams(use_tc_tiling_on_sc=False),
        scratch_types=[
            pltpu.VMEM((G, RPW), jnp.int32),
            pltpu.VMEM((G * RPW,), jnp.float32),
            pltpu.VMEM((16,), jnp.float32),
            pltpu.VMEM((RPW,), jnp.float32),
            pltpu.SemaphoreType.DMA,
            pltpu.SemaphoreType.DMA,
            pltpu.SemaphoreType.DMA,
            pltpu.SemaphoreType.DMA,
            pltpu.SemaphoreType.DMA,
        ],
    )
    def body(*refs):
        tabs = refs[:G]
        (idx_hbm, bias_hbm, out_hbm, idx_v, buf, bias_v, out_v, sem_idx,
         s0, s1, s2, s3) = refs[G:]
        sems = (s0, s1, s2, s3)
        wid = lax.axis_index("s") * NC + lax.axis_index("c")
        base = wid * RPW
        pltpu.sync_copy(bias_hbm, bias_v)
        for f in range(G):
            pltpu.async_copy(
                idx_hbm.at[f, pl.ds(base, RPW)], idx_v.at[f], sem_idx
            )
        for f in range(G):
            pltpu.make_async_copy(
                idx_hbm.at[f, pl.ds(base, RPW)], idx_v.at[f], sem_idx
            ).wait()

        bvec = bias_v[...]

        def chunk(f, c):
            return (
                tabs[f].at[idx_v.at[f, pl.ds(c * CH, CH)]],
                buf.at[pl.ds(f * RPW + c * CH, CH)],
            )

        def fire(c):
            for f in range(G):
                src, dst = chunk(f, c)
                pltpu.async_copy(src, dst, sems[c])

        def drain(c):
            for f in range(G):
                src, dst = chunk(f, c)
                pltpu.make_async_copy(src, dst, sems[c]).wait()

        def reduce_block(c):
            for v in range(CH // 16):
                acc = bvec
                for f in range(G):
                    acc = acc + buf[pl.ds(f * RPW + c * CH + v * 16, 16)]
                out_v[pl.ds(c * CH + v * 16, 16)] = acc

        fire(0)
        fire(1)
        for c in range(CPF):
            drain(c)
            if c + 2 < CPF:
                fire(c + 2)
            reduce_block(c)
        pltpu.sync_copy(out_v, out_hbm.at[pl.ds(base, RPW)])

    return body(*table_rows, idx_t, bias16)


def kernel(x, tables, bias):
    table_rows = [tables[f, :, 0] for f in range(F)]
    idx_t = x.T.astype(jnp.int32)
    bias16 = jnp.broadcast_to(bias.astype(jnp.float32), (16,))
    zero16 = jnp.zeros((16,), jnp.float32)
    p0 = _sc_gather_sum(table_rows[:G], idx_t[:G], bias16)
    p1 = _sc_gather_sum(table_rows[G:], idx_t[G:], zero16)
    return (p0 + p1).reshape(B, 1)
